# baseline (speedup 1.0000x reference)
import jax
import jax.numpy as jnp
from jax import lax
from jax.experimental import pallas as pl
from jax.experimental.pallas import tpu as pltpu

_B = 16384
_K = 128
_N = 64
_SIZES = (2560, 2560, 2560, 2560, 2048, 1536, 1536, 1024)
_OFFS = tuple(sum(_SIZES[:c]) for c in range(len(_SIZES)))
_NC = len(_SIZES)


def _body(x_hbm, wt_hbm, o_hbm, x_vmem, o_vmem, wt_vmem, in_sems, out_sems,
          w_sem):
    def in_copy(c):
        sl = pl.ds(_OFFS[c], _SIZES[c])
        return pltpu.make_async_copy(x_hbm.at[sl, :], x_vmem.at[sl, :],
                                     in_sems.at[c])

    def out_copy(c):
        sl = pl.ds(_OFFS[c], _SIZES[c])
        return pltpu.make_async_copy(o_vmem.at[:, sl], o_hbm.at[:, sl],
                                     out_sems.at[c])

    for c in range(_NC):
        in_copy(c).start()
    w_copy = pltpu.make_async_copy(wt_hbm, wt_vmem, w_sem)
    w_copy.start()
    w_copy.wait()
    for c in range(_NC):
        in_copy(c).wait()
        sl = pl.ds(_OFFS[c], _SIZES[c])
        o_vmem[:, sl] = lax.dot_general(
            wt_vmem[...], x_vmem[sl, :],
            (((1,), (1,)), ((), ())),
            preferred_element_type=jnp.float32)
        out_copy(c).start()
    for c in range(_NC):
        out_copy(c).wait()


@jax.jit
def _matmul_t(x, Wt):
    return pl.pallas_call(
        _body,
        in_specs=[
            pl.BlockSpec(memory_space=pl.ANY),
            pl.BlockSpec(memory_space=pl.ANY),
        ],
        out_specs=pl.BlockSpec(memory_space=pl.ANY),
        out_shape=jax.ShapeDtypeStruct((_N, _B), jnp.float32),
        scratch_shapes=[
            pltpu.VMEM((_B, _K), jnp.float32),
            pltpu.VMEM((_N, _B), jnp.float32),
            pltpu.VMEM((_N, _K), jnp.float32),
            pltpu.SemaphoreType.DMA((_NC,)),
            pltpu.SemaphoreType.DMA((_NC,)),
            pltpu.SemaphoreType.DMA,
        ],
    )(x, Wt)


def kernel(x, W):
    x = x.reshape(x.shape[0], -1)
    return _matmul_t(x, W.T).T


# speedup vs baseline: 1.0132x; 1.0132x over previous
import jax
import jax.numpy as jnp
from jax import lax
from jax.experimental import pallas as pl
from jax.experimental.pallas import tpu as pltpu

_B = 16384
_K = 128
_N = 64
_SIZES = (2560, 2560, 2560, 2560, 2048, 1536, 1536, 1024)
_OFFS = tuple(sum(_SIZES[:c]) for c in range(len(_SIZES)))
_NC = len(_SIZES)


def _body(x_hbm, wt_hbm, o_hbm, x_vmem, o_vmem, wt_vmem, in_sems, out_sems,
          w_sem):
    def in_copy(c):
        sl = pl.ds(_OFFS[c], _SIZES[c])
        return pltpu.make_async_copy(x_hbm.at[sl, :], x_vmem.at[sl, :],
                                     in_sems.at[c])

    def out_copy(c):
        sl = pl.ds(_OFFS[c], _SIZES[c])
        return pltpu.make_async_copy(o_vmem.at[:, sl], o_hbm.at[:, sl],
                                     out_sems.at[c])

    w_copy = pltpu.make_async_copy(wt_hbm, wt_vmem, w_sem)
    w_copy.start()
    for c in range(_NC):
        in_copy(c).start()
    w_copy.wait()
    for c in range(_NC):
        in_copy(c).wait()
        sl = pl.ds(_OFFS[c], _SIZES[c])
        o_vmem[:, sl] = lax.dot_general(
            wt_vmem[...], x_vmem[sl, :],
            (((1,), (1,)), ((), ())),
            preferred_element_type=jnp.float32)
        out_copy(c).start()
    for c in range(_NC):
        out_copy(c).wait()


@jax.jit
def _matmul_t(x, Wt):
    return pl.pallas_call(
        _body,
        in_specs=[
            pl.BlockSpec(memory_space=pl.ANY),
            pl.BlockSpec(memory_space=pl.ANY),
        ],
        out_specs=pl.BlockSpec(memory_space=pl.ANY),
        out_shape=jax.ShapeDtypeStruct((_N, _B), jnp.float32),
        scratch_shapes=[
            pltpu.VMEM((_B, _K), jnp.float32),
            pltpu.VMEM((_N, _B), jnp.float32),
            pltpu.VMEM((_N, _K), jnp.float32),
            pltpu.SemaphoreType.DMA((_NC,)),
            pltpu.SemaphoreType.DMA((_NC,)),
            pltpu.SemaphoreType.DMA,
        ],
    )(x, Wt)


def kernel(x, W):
    x = x.reshape(x.shape[0], -1)
    return _matmul_t(x, W.T).T


# speedup vs baseline: 1.1120x; 1.0975x over previous
import jax
import jax.numpy as jnp
from jax import lax
from jax.experimental import pallas as pl
from jax.experimental.pallas import tpu as pltpu

_B = 16384
_K = 128
_N = 64
_SIZES = (3584, 3072, 2560, 2048, 1536, 1024, 1024, 1024, 512)
_OFFS = tuple(sum(_SIZES[:c]) for c in range(len(_SIZES)))
_NC = len(_SIZES)


def _body(x_hbm, wt_ref, o_hbm, x_vmem, o_vmem, in_sems, out_sems):
    def in_copy(c):
        sl = pl.ds(_OFFS[c], _SIZES[c])
        return pltpu.make_async_copy(x_hbm.at[sl, :], x_vmem.at[sl, :],
                                     in_sems.at[c])

    def out_copy(c):
        sl = pl.ds(_OFFS[c], _SIZES[c])
        return pltpu.make_async_copy(o_vmem.at[:, sl], o_hbm.at[:, sl],
                                     out_sems.at[c])

    for c in range(_NC):
        in_copy(c).start()
    for c in range(_NC):
        in_copy(c).wait()
        sl = pl.ds(_OFFS[c], _SIZES[c])
        o_vmem[:, sl] = lax.dot_general(
            wt_ref[...], x_vmem[sl, :],
            (((1,), (1,)), ((), ())),
            preferred_element_type=jnp.float32)
        out_copy(c).start()
    for c in range(_NC):
        out_copy(c).wait()


@jax.jit
def _matmul_t(x, Wt):
    return pl.pallas_call(
        _body,
        in_specs=[
            pl.BlockSpec(memory_space=pl.ANY),
            pl.BlockSpec((_N, _K), lambda: (0, 0)),
        ],
        out_specs=pl.BlockSpec(memory_space=pl.ANY),
        out_shape=jax.ShapeDtypeStruct((_N, _B), jnp.float32),
        scratch_shapes=[
            pltpu.VMEM((_B, _K), jnp.float32),
            pltpu.VMEM((_N, _B), jnp.float32),
            pltpu.SemaphoreType.DMA((_NC,)),
            pltpu.SemaphoreType.DMA((_NC,)),
        ],
    )(x, Wt)


def kernel(x, W):
    x = x.reshape(x.shape[0], -1)
    return _matmul_t(x, W.T).T


# speedup vs baseline: 1.1279x; 1.0143x over previous
import jax
import jax.numpy as jnp
from jax import lax
from jax.experimental import pallas as pl
from jax.experimental.pallas import tpu as pltpu

_B = 16384
_K = 128
_N = 64
_SIZES = (2560, 2560, 2560, 2560, 2048, 1536, 1536, 1024)
_OFFS = tuple(sum(_SIZES[:c]) for c in range(len(_SIZES)))
_NC = len(_SIZES)


def _body(x_hbm, wt_ref, o_hbm, x_vmem, o_vmem, in_sems, out_sems):
    def in_copy(c):
        sl = pl.ds(_OFFS[c], _SIZES[c])
        return pltpu.make_async_copy(x_hbm.at[sl, :], x_vmem.at[sl, :],
                                     in_sems.at[c])

    def out_copy(c):
        sl = pl.ds(_OFFS[c], _SIZES[c])
        return pltpu.make_async_copy(o_vmem.at[:, sl], o_hbm.at[:, sl],
                                     out_sems.at[c])

    for c in range(_NC):
        in_copy(c).start()
    for c in range(_NC):
        in_copy(c).wait()
        sl = pl.ds(_OFFS[c], _SIZES[c])
        o_vmem[:, sl] = lax.dot_general(
            wt_ref[...], x_vmem[sl, :],
            (((1,), (1,)), ((), ())),
            preferred_element_type=jnp.float32)
        out_copy(c).start()
    for c in range(_NC):
        out_copy(c).wait()


@jax.jit
def _matmul_t(x, Wt):
    return pl.pallas_call(
        _body,
        in_specs=[
            pl.BlockSpec(memory_space=pl.ANY),
            pl.BlockSpec((_N, _K), lambda: (0, 0)),
        ],
        out_specs=pl.BlockSpec(memory_space=pl.ANY),
        out_shape=jax.ShapeDtypeStruct((_N, _B), jnp.float32),
        scratch_shapes=[
            pltpu.VMEM((_B, _K), jnp.float32),
            pltpu.VMEM((_N, _B), jnp.float32),
            pltpu.SemaphoreType.DMA((_NC,)),
            pltpu.SemaphoreType.DMA((_NC,)),
        ],
    )(x, Wt)


def kernel(x, W):
    x = x.reshape(x.shape[0], -1)
    return _matmul_t(x, W.T).T
